# Initial kernel scaffold; baseline (speedup 1.0000x reference)
import jax
import jax.numpy as jnp
from jax.experimental import pallas as pl


def kernel(t01, pos01, index, sigma_norms):
    raise NotImplementedError("write your pallas kernel here")


# speedup vs baseline: 1.6090x; 1.6090x over previous
import functools
import math

import jax
import jax.numpy as jnp
from jax import lax
from jax.experimental import pallas as pl
from jax.experimental.pallas import tpu as pltpu
from jax.experimental.pallas import tpu_sc as plsc

SCALE_POS = 2.0 * math.pi
TF = 2.0
KWN = 13
N_SIGMAS = 2000
SEGS = 100000
SEGS_PAD = 100096


def _sc_mesh():
    return plsc.VectorSubcoreMesh(core_axis_name="c", subcore_axis_name="s")


@jax.jit
def _seg_sum(vals, idx, zeros):
    n = idx.shape[0]
    info = plsc.get_sparse_core_info()
    nc, ns = info.num_cores, info.num_subcores
    nw = nc * ns
    per_w = n // nw
    ch = 80
    n_ch = per_w // ch
    assert per_w % ch == 0 and n % nw == 0

    @functools.partial(
        pl.kernel,
        mesh=_sc_mesh(),
        compiler_params=pltpu.CompilerParams(use_tc_tiling_on_sc=False),
        out_type=jax.ShapeDtypeStruct((nc, SEGS_PAD, 16), jnp.float32),
        scratch_types=[
            pltpu.VMEM((ch,), jnp.int32),
            pltpu.VMEM((ch, 16), jnp.float32),
            pltpu.VMEM_SHARED((SEGS_PAD, 16), jnp.float32),
        ],
    )
    def k(vals_hbm, idx_hbm, zeros_hbm, out_hbm, idx_v, vals_v, acc_sh):
        cid = lax.axis_index("c")
        sid = lax.axis_index("s")
        wid = sid * nc + cid

        @pl.when(sid == 0)
        def _():
            pltpu.sync_copy(zeros_hbm, acc_sh)

        plsc.subcore_barrier()

        def body(j, carry):
            off = wid * per_w + j * ch
            pltpu.sync_copy(idx_hbm.at[pl.ds(off, ch)], idx_v)
            pltpu.sync_copy(vals_hbm.at[pl.ds(off, ch)], vals_v)
            pltpu.sync_copy(vals_v, acc_sh.at[idx_v], add=True)
            return carry

        lax.fori_loop(0, n_ch, body, 0)
        plsc.subcore_barrier()
        rows = SEGS_PAD // ns
        pltpu.sync_copy(
            acc_sh.at[pl.ds(sid * rows, rows)],
            out_hbm.at[cid, pl.ds(sid * rows, rows)],
        )

    return k(vals, idx, zeros)


@jax.jit
def _sc_gather(table, idx):
    n = idx.shape[0]
    info = plsc.get_sparse_core_info()
    nc, ns = info.num_cores, info.num_subcores
    nw = nc * ns
    per_w = n // nw
    ch = 80
    n_ch = per_w // ch
    assert per_w % ch == 0 and n % nw == 0

    @functools.partial(
        pl.kernel,
        mesh=_sc_mesh(),
        compiler_params=pltpu.CompilerParams(use_tc_tiling_on_sc=False),
        out_type=jax.ShapeDtypeStruct((n, 16), jnp.float32),
        scratch_types=[
            pltpu.VMEM((ch,), jnp.int32),
            pltpu.VMEM((ch, 16), jnp.float32),
            pltpu.SemaphoreType.DMA,
        ],
    )
    def k(table_hbm, idx_hbm, out_hbm, idx_v, rows_v, sem):
        wid = lax.axis_index("s") * nc + lax.axis_index("c")

        def body(j, carry):
            off = wid * per_w + j * ch
            pltpu.sync_copy(idx_hbm.at[pl.ds(off, ch)], idx_v)
            pltpu.async_copy(table_hbm.at[idx_v], rows_v, sem).wait()
            pltpu.sync_copy(rows_v, out_hbm.at[pl.ds(off, ch)])
            return carry

        lax.fori_loop(0, n_ch, body, 0)

    return k(table, idx)


_BR = 4096


def _dlogp(d, sigma):
    num = jnp.zeros_like(d)
    den = jnp.zeros_like(d)
    for i in range(-KWN, KWN + 1):
        s = d + SCALE_POS * i
        e = jnp.exp(-(s * s) / (2.0 * sigma * sigma))
        num = num + (-(s) / (sigma * sigma)) * e
        den = den + e
    return num / (den + 1e-30)


def _math1_body(tb, posf, evf, erf, svf, srf, cntf, vt_o, pt_o, tpt_o):
    t = TF * tb[...]
    cnt = jnp.maximum(cntf[...], 1.0)
    eps_v = evf[...] - svf[...] / cnt
    sigma_v = jnp.sqrt(1.0 - jnp.exp(-2.0 * t) + 1e-6)
    v_t = eps_v * sigma_v
    pos = SCALE_POS * (posf[...] - jnp.round(posf[...]))
    prefactor = (1.0 - jnp.exp(-t)) / (1.0 + jnp.exp(-t))
    mu_r = prefactor * v_t
    sigma_r = jnp.sqrt(2.0 * t + 8.0 / (1.0 + jnp.exp(t)) - 4.0 + 1e-6)
    eps_r = erf[...] - srf[...] / cnt
    x = mu_r + sigma_r * eps_r
    r = x - jnp.round(x)
    mu_rw = mu_r - jnp.round(mu_r)
    y = pos + r
    pos_t = y - jnp.round(y)
    tpt = prefactor * _dlogp(r - mu_rw, sigma_r)
    vt_o[...] = v_t / SCALE_POS
    pt_o[...] = pos_t / SCALE_POS
    tpt_o[...] = tpt


def _math2_body(tb, tptf, smf, cntf, snf, out_o):
    t = TF * tb[...]
    prefactor = (1.0 - jnp.exp(-t)) / (1.0 + jnp.exp(-t))
    cnt = jnp.maximum(cntf[...], 1.0)
    centered = tptf[...] - smf[...] / cnt
    out_o[...] = centered / prefactor / jnp.sqrt(snf[...])


def _flat_call(body, n_out, args):
    r = args[0].shape[0]
    grid = pl.cdiv(r, _BR)
    bs = pl.BlockSpec((_BR, 128), lambda i: (i, 0))
    return pl.pallas_call(
        body,
        grid=(grid,),
        in_specs=[bs] * len(args),
        out_specs=[bs] * n_out,
        out_shape=[jax.ShapeDtypeStruct((r, 128), jnp.float32)] * n_out,
    )(*args)


def kernel(t01, pos01, index, sigma_norms):
    n = index.shape[0]
    rkey = jax.random.key(1)
    ev = jax.random.normal(jax.random.fold_in(rkey, 1), (n, 3), jnp.float32)
    er = jax.random.normal(jax.random.fold_in(rkey, 2), (n, 3), jnp.float32)
    idx = index.astype(jnp.int32)
    zeros16 = jnp.zeros((SEGS_PAD, 16), jnp.float32)

    vals1 = jnp.concatenate(
        [ev, er, jnp.ones((n, 1), jnp.float32), jnp.zeros((n, 9), jnp.float32)], axis=1
    )
    part1 = _seg_sum(vals1, idx, zeros16)
    sums1 = part1[0] + part1[1]
    g1 = _sc_gather(sums1, idx)

    isn = jnp.round(t01[:, 0] * N_SIGMAS).astype(jnp.int32) - 1
    sn_table = jnp.broadcast_to(sigma_norms[:, None], (N_SIGMAS, 16))
    g3 = _sc_gather(sn_table, isn)

    def flat(a):
        return jnp.reshape(a, (-1, 128))

    def bcast(col):
        return flat(jnp.broadcast_to(col, (n, 3)))

    tb = bcast(t01)
    cntf = bcast(g1[:, 6:7])
    vt_f, pt_f, tpt_f = _flat_call(
        _math1_body,
        3,
        (tb, flat(pos01), flat(ev), flat(er), flat(g1[:, 0:3]), flat(g1[:, 3:6]), cntf),
    )

    tpt = jnp.reshape(tpt_f, (n, 3))
    vals2 = jnp.concatenate([tpt, jnp.zeros((n, 13), jnp.float32)], axis=1)
    part2 = _seg_sum(vals2, idx, zeros16)
    sums2 = part2[0] + part2[1]
    g2 = _sc_gather(sums2, idx)

    (target_f,) = _flat_call(
        _math2_body, 1, (tb, tpt_f, flat(g2[:, 0:3]), cntf, bcast(g3[:, 0:1]))
    )

    return (
        jnp.reshape(vt_f, (n, 3)),
        jnp.reshape(pt_f, (n, 3)),
        jnp.reshape(target_f, (n, 3)),
    )
